# Initial kernel scaffold; baseline (speedup 1.0000x reference)
import jax
import jax.numpy as jnp
from jax.experimental import pallas as pl


def kernel(x, edge_index, edge_type, Win, b_in, comp_s, basis_s, root_s, bias_s, comp_t, basis_t, root_t, bias_t, Wf, bf, gamma, beta, W1, b1, W2, b2):
    raise NotImplementedError("write your pallas kernel here")


# speedup vs baseline: 6.0758x; 6.0758x over previous
import functools

import jax
import jax.numpy as jnp
from jax import lax
from jax.experimental import pallas as pl
from jax.experimental.pallas import tpu as pltpu

_N = 10000
_E = 320000
_H = 128
_O = 32
_L = 3
_RS = 7
_RT = 3
_R = _RS + _RT


def _layernorm(x, g, b):
    m = jnp.mean(x, axis=-1, keepdims=True)
    v = jnp.var(x, axis=-1, keepdims=True)
    return (x - m) / jnp.sqrt(v + 1e-5) * g + b


def _head_body(h_ref, w1_ref, b1_ref, w2_ref, b2_ref, o_ref):
    h = h_ref[...]
    t = jnp.maximum(jnp.dot(h, w1_ref[...], preferred_element_type=jnp.float32)
                    + b1_ref[...], 0.0)
    o_ref[...] = jnp.dot(t, w2_ref[...], preferred_element_type=jnp.float32) + b2_ref[...]


def _head(h, W1, b1, W2, b2):
    n = h.shape[0]
    blk = 512
    npad = ((n + blk - 1) // blk) * blk
    hp = jnp.pad(h, ((0, npad - n), (0, 0)))
    out = pl.pallas_call(
        _head_body,
        grid=(npad // blk,),
        in_specs=[
            pl.BlockSpec((blk, _H), lambda i: (i, 0)),
            pl.BlockSpec((_H, _H), lambda i: (0, 0)),
            pl.BlockSpec((_H,), lambda i: (0,)),
            pl.BlockSpec((_H, _O), lambda i: (0, 0)),
            pl.BlockSpec((_O,), lambda i: (0,)),
        ],
        out_specs=pl.BlockSpec((blk, _O), lambda i: (i, 0)),
        out_shape=jax.ShapeDtypeStruct((npad, _O), jnp.float32),
    )(hp, W1, b1, W2, b2)
    return out[:n]


def kernel(x, edge_index, edge_type, Win, b_in, comp_s, basis_s, root_s,
           bias_s, comp_t, basis_t, root_t, bias_t, Wf, bf, gamma, beta,
           W1, b1, W2, b2):
    n = x.shape[0]
    src = edge_index[0]
    dst = edge_index[1]
    et = edge_type

    keys = dst * _R + et
    cnt = jax.ops.segment_sum(jnp.ones((_E,), jnp.float32), keys,
                              num_segments=n * _R)
    w = 1.0 / jnp.maximum(cnt[keys], 1.0)

    sm = et < _RS
    out_idx = dst + jnp.where(sm, 0, n)
    gath_idx = et * n + src

    h = x @ Win + b_in
    for i in range(_L):
        Ws = jnp.einsum('rb,bio->rio', comp_s[i], basis_s[i])
        Wt = jnp.einsum('rb,bio->rio', comp_t[i], basis_t[i])
        Wall = jnp.concatenate([Ws, Wt], axis=0)
        hr = jnp.einsum('ni,rio->rno', h, Wall)
        hr = hr.reshape(_R * n, _H)
        msg = hr[gath_idx] * w[:, None]
        agg = jax.ops.segment_sum(msg, out_idx, num_segments=2 * n)
        hs = agg[:n] + h @ root_s[i] + bias_s[i]
        ht = agg[n:] + h @ root_t[i] + bias_t[i]
        hn = jnp.concatenate([hs, ht], axis=-1) @ Wf[i] + bf[i]
        hn = jax.nn.relu(_layernorm(hn, gamma[i], beta[i]))
        h = h + hn if i > 0 else hn
    return _head(h, W1, b1, W2, b2)


# speedup vs baseline: 14.5922x; 2.4017x over previous
import functools

import jax
import jax.numpy as jnp
from jax import lax
from jax.experimental import pallas as pl
from jax.experimental.pallas import tpu as pltpu
from jax.experimental.pallas import tpu_sc as plsc

_N = 10000
_E = 320000
_H = 128
_O = 32
_L = 3
_RS = 7
_RT = 3
_R = _RS + _RT

_NC = 2
_NS = 16
_CH = 400
_EPT = _E // _NS
_NCH = _EPT // _CH
_HH = _H // 2

_ACC_PAD = 20096
_RPT = _ACC_PAD // _NS
_CNT_PT = 6272
_CNT_ROWS = _CNT_PT * _NS
_EPW = _E // (_NC * _NS)
_NCHW = _EPW // _CH

_sc_mesh = plsc.VectorSubcoreMesh(core_axis_name="c", subcore_axis_name="s")


@functools.partial(
    pl.kernel,
    out_type=jax.ShapeDtypeStruct((_E,), jnp.float32),
    mesh=_sc_mesh,
    compiler_params=pltpu.CompilerParams(use_tc_tiling_on_sc=False),
    scratch_types=[
        pltpu.VMEM_SHARED((_CNT_ROWS,), jnp.float32),
        pltpu.VMEM((_CH,), jnp.int32),
        pltpu.VMEM((_CH,), jnp.float32),
        pltpu.VMEM((_CH,), jnp.float32),
        pltpu.VMEM((_CH,), jnp.float32),
        pltpu.SemaphoreType.DMA,
    ],
)
def _prep(keys_hbm, ones_hbm, zc_hbm, w_hbm,
          cnt, ki_v, ones_v, cv_v, wv_v, sem):
    c = lax.axis_index("c")
    s = lax.axis_index("s")
    pltpu.sync_copy(zc_hbm, cnt.at[pl.ds(s * _CNT_PT, _CNT_PT)])
    pltpu.sync_copy(ones_hbm, ones_v)
    plsc.subcore_barrier()

    def count_body(ci, _):
        e0 = s * _EPT + ci * _CH
        pltpu.sync_copy(keys_hbm.at[pl.ds(e0, _CH)], ki_v)
        pltpu.sync_copy(ones_v, cnt.at[ki_v], add=True)
        return 0

    lax.fori_loop(0, _NCH, count_body, 0)
    plsc.subcore_barrier()

    def w_body(ci, _):
        e0 = (c * _NS + s) * _EPW + ci * _CH
        pltpu.sync_copy(keys_hbm.at[pl.ds(e0, _CH)], ki_v)
        pltpu.async_copy(cnt.at[ki_v], cv_v, sem).wait()

        def div_body(j, _):
            cv = cv_v[pl.ds(j * 16, 16)]
            wv_v[pl.ds(j * 16, 16)] = 1.0 / cv
            return 0

        lax.fori_loop(0, _CH // 16, div_body, 0)
        pltpu.sync_copy(wv_v, w_hbm.at[pl.ds(e0, _CH)])
        return 0

    lax.fori_loop(0, _NCHW, w_body, 0)


@functools.partial(
    pl.kernel,
    out_type=jax.ShapeDtypeStruct((_NC, _ACC_PAD, _HH), jnp.float32),
    mesh=_sc_mesh,
    compiler_params=pltpu.CompilerParams(use_tc_tiling_on_sc=False),
    scratch_types=[
        pltpu.VMEM_SHARED((_ACC_PAD, _HH), jnp.float32),
        pltpu.VMEM((_CH,), jnp.int32),
        pltpu.VMEM((_CH,), jnp.int32),
        pltpu.VMEM((_CH, 16), jnp.float32),
        pltpu.VMEM((_CH, _HH), jnp.float32),
        pltpu.SemaphoreType.DMA,
    ],
)
def _edge_pass(hr_hbm, gi_hbm, si_hbm, w_hbm, za_hbm, out_hbm,
               acc, gi_v, si_v, w_v, rows_v, sem):
    c = lax.axis_index("c")
    s = lax.axis_index("s")
    pltpu.sync_copy(za_hbm, acc.at[pl.ds(s * _RPT, _RPT)])
    plsc.subcore_barrier()

    def chunk_body(ci, _):
        e0 = s * _EPT + ci * _CH
        pltpu.sync_copy(gi_hbm.at[pl.ds(e0, _CH)], gi_v)
        pltpu.sync_copy(si_hbm.at[pl.ds(e0, _CH)], si_v)
        pltpu.sync_copy(w_hbm.at[pl.ds(e0, _CH)], w_v)
        pltpu.async_copy(hr_hbm.at[c].at[gi_v], rows_v, sem).wait()

        def scale_body(i, _):
            wv = w_v[i, :]
            for k in range(_HH // 16):
                sl = pl.ds(k * 16, 16)
                rows_v[i, sl] = rows_v[i, sl] * wv
            return 0

        lax.fori_loop(0, _CH, scale_body, 0)
        pltpu.sync_copy(rows_v, acc.at[si_v], add=True)
        return 0

    lax.fori_loop(0, _NCH, chunk_body, 0)
    plsc.subcore_barrier()
    pltpu.sync_copy(acc.at[pl.ds(s * _RPT, _RPT)],
                    out_hbm.at[c].at[pl.ds(s * _RPT, _RPT)])


_BN = 400


def _proj_body(x_ref, w_ref, b_ref, o_ref):
    o_ref[...] = (jnp.dot(x_ref[...], w_ref[...],
                          preferred_element_type=jnp.float32) + b_ref[...])


def _proj(x, Win, b_in):
    return pl.pallas_call(
        _proj_body,
        grid=(_N // _BN,),
        in_specs=[
            pl.BlockSpec((_BN, _H), lambda i: (i, 0)),
            pl.BlockSpec((_H, _H), lambda i: (0, 0)),
            pl.BlockSpec((1, _H), lambda i: (0, 0)),
        ],
        out_specs=pl.BlockSpec((_BN, _H), lambda i: (i, 0)),
        out_shape=jax.ShapeDtypeStruct((_N, _H), jnp.float32),
    )(x, Win, b_in.reshape(1, _H))


def _hr_body(h_ref, w_ref, o_ref):
    o_ref[...] = jnp.dot(h_ref[...], w_ref[0, 0],
                         preferred_element_type=jnp.float32)[None, None]


def _hr_halves(h, Wall_sp):
    return pl.pallas_call(
        _hr_body,
        grid=(_NC, _R, _N // _BN),
        in_specs=[
            pl.BlockSpec((_BN, _H), lambda c, r, i: (i, 0)),
            pl.BlockSpec((1, 1, _H, _HH), lambda c, r, i: (c, r, 0, 0)),
        ],
        out_specs=pl.BlockSpec((1, 1, _BN, _HH), lambda c, r, i: (c, r, i, 0)),
        out_shape=jax.ShapeDtypeStruct((_NC, _R, _N, _HH), jnp.float32),
    )(h, Wall_sp)


def _fuse_body(first, s0_ref, s1_ref, t0_ref, t1_ref, h_ref, rs_ref, rt_ref,
               bs_ref, bt_ref, wft_ref, wfb_ref, bf_ref, g_ref, be_ref, o_ref):
    h = h_ref[...]
    hs = jnp.concatenate([s0_ref[...], s1_ref[...]], axis=-1) + \
        jnp.dot(h, rs_ref[...], preferred_element_type=jnp.float32) + bs_ref[...]
    ht = jnp.concatenate([t0_ref[...], t1_ref[...]], axis=-1) + \
        jnp.dot(h, rt_ref[...], preferred_element_type=jnp.float32) + bt_ref[...]
    hn = (jnp.dot(hs, wft_ref[...], preferred_element_type=jnp.float32)
          + jnp.dot(ht, wfb_ref[...], preferred_element_type=jnp.float32)
          + bf_ref[...])
    m = jnp.mean(hn, axis=-1, keepdims=True)
    d = hn - m
    v = jnp.mean(d * d, axis=-1, keepdims=True)
    hn = d * lax.rsqrt(v + 1e-5) * g_ref[...] + be_ref[...]
    hn = jnp.maximum(hn, 0.0)
    o_ref[...] = hn if first else h + hn


def _fuse(first, aggs0, aggs1, aggt0, aggt1, h, roots, roott, bias_s, bias_t,
          wf_top, wf_bot, bf_i, gamma_i, beta_i):
    row = lambda a: a.reshape(1, _H)
    half = pl.BlockSpec((_BN, _HH), lambda i: (i, 0))
    full = pl.BlockSpec((_BN, _H), lambda i: (i, 0))
    wspec = pl.BlockSpec((_H, _H), lambda i: (0, 0))
    bspec = pl.BlockSpec((1, _H), lambda i: (0, 0))
    return pl.pallas_call(
        functools.partial(_fuse_body, first),
        grid=(_N // _BN,),
        in_specs=[half, half, half, half, full, wspec, wspec, bspec, bspec,
                  wspec, wspec, bspec, bspec, bspec],
        out_specs=full,
        out_shape=jax.ShapeDtypeStruct((_N, _H), jnp.float32),
    )(aggs0, aggs1, aggt0, aggt1, h, roots, roott, row(bias_s), row(bias_t),
      wf_top, wf_bot, row(bf_i), row(gamma_i), row(beta_i))


def _head_body(h_ref, w1_ref, b1_ref, w2_ref, b2_ref, o_ref):
    t = jnp.maximum(jnp.dot(h_ref[...], w1_ref[...],
                            preferred_element_type=jnp.float32) + b1_ref[...], 0.0)
    o_ref[...] = jnp.dot(t, w2_ref[...],
                         preferred_element_type=jnp.float32) + b2_ref[...]


def _head(h, W1, b1, W2, b2):
    return pl.pallas_call(
        _head_body,
        grid=(_N // _BN,),
        in_specs=[
            pl.BlockSpec((_BN, _H), lambda i: (i, 0)),
            pl.BlockSpec((_H, _H), lambda i: (0, 0)),
            pl.BlockSpec((1, _H), lambda i: (0, 0)),
            pl.BlockSpec((_H, _O), lambda i: (0, 0)),
            pl.BlockSpec((1, _O), lambda i: (0, 0)),
        ],
        out_specs=pl.BlockSpec((_BN, _O), lambda i: (i, 0)),
        out_shape=jax.ShapeDtypeStruct((_N, _O), jnp.float32),
    )(h, W1, b1.reshape(1, _H), W2, b2.reshape(1, _O))


def kernel(x, edge_index, edge_type, Win, b_in, comp_s, basis_s, root_s,
           bias_s, comp_t, basis_t, root_t, bias_t, Wf, bf, gamma, beta,
           W1, b1, W2, b2):
    src = edge_index[0]
    dst = edge_index[1]
    et = edge_type

    keys = dst * _R + et
    gidx = et * _N + src
    sidx = dst + jnp.where(et < _RS, 0, _N)

    ones = jnp.ones((_CH,), jnp.float32)
    zc = jnp.zeros((_CNT_PT,), jnp.float32)
    za = jnp.zeros((_RPT, _HH), jnp.float32)

    w = _prep(keys, ones, zc)
    w16 = jnp.broadcast_to(w[:, None], (_E, 16))

    h = _proj(x, Win, b_in)
    for i in range(_L):
        Ws = jnp.einsum('rb,bio->rio', comp_s[i], basis_s[i])
        Wt = jnp.einsum('rb,bio->rio', comp_t[i], basis_t[i])
        Wall = jnp.concatenate([Ws, Wt], axis=0)
        Wall_sp = Wall.reshape(_R, _H, _NC, _HH).transpose(2, 0, 1, 3)
        hr = _hr_halves(h, Wall_sp).reshape(_NC, _R * _N, _HH)
        agg = _edge_pass(hr, gidx, sidx, w16, za)
        h = _fuse(i == 0,
                  agg[0, :_N], agg[1, :_N], agg[0, _N:], agg[1, _N:],
                  h, root_s[i], root_t[i], bias_s[i], bias_t[i],
                  Wf[i][:_H], Wf[i][_H:], bf[i], gamma[i], beta[i])
    return _head(h, W1, b1, W2, b2)


# speedup vs baseline: 18.7722x; 1.2865x over previous
import functools

import jax
import jax.numpy as jnp
from jax import lax
from jax.experimental import pallas as pl
from jax.experimental.pallas import tpu as pltpu
from jax.experimental.pallas import tpu_sc as plsc

_N = 10000
_E = 320000
_H = 128
_O = 32
_L = 3
_RS = 7
_RT = 3
_R = _RS + _RT

_NC = 2
_NS = 16
_CH = 400
_EPT = _E // _NS
_NCH = _EPT // _CH
_HH = _H // 2

_ACC_PAD = 20096
_RPT = _ACC_PAD // _NS
_CNT_PT = 6272
_CNT_ROWS = _CNT_PT * _NS
_EPW = _E // (_NC * _NS)
_NCHW = _EPW // _CH

_sc_mesh = plsc.VectorSubcoreMesh(core_axis_name="c", subcore_axis_name="s")


@functools.partial(
    pl.kernel,
    out_type=jax.ShapeDtypeStruct((_E,), jnp.float32),
    mesh=_sc_mesh,
    compiler_params=pltpu.CompilerParams(use_tc_tiling_on_sc=False),
    scratch_types=[
        pltpu.VMEM_SHARED((_CNT_ROWS,), jnp.float32),
        pltpu.VMEM((_CH,), jnp.int32),
        pltpu.VMEM((_CH,), jnp.float32),
        pltpu.VMEM((_CH,), jnp.float32),
        pltpu.VMEM((_CH,), jnp.float32),
        pltpu.SemaphoreType.DMA,
    ],
)
def _prep(keys_hbm, ones_hbm, zc_hbm, w_hbm,
          cnt, ki_v, ones_v, cv_v, wv_v, sem):
    c = lax.axis_index("c")
    s = lax.axis_index("s")
    pltpu.sync_copy(zc_hbm, cnt.at[pl.ds(s * _CNT_PT, _CNT_PT)])
    pltpu.sync_copy(ones_hbm, ones_v)
    plsc.subcore_barrier()

    def count_body(ci, _):
        e0 = s * _EPT + ci * _CH
        pltpu.sync_copy(keys_hbm.at[pl.ds(e0, _CH)], ki_v)
        pltpu.sync_copy(ones_v, cnt.at[ki_v], add=True)
        return 0

    lax.fori_loop(0, _NCH, count_body, 0)
    plsc.subcore_barrier()

    def w_body(ci, _):
        e0 = (c * _NS + s) * _EPW + ci * _CH
        pltpu.sync_copy(keys_hbm.at[pl.ds(e0, _CH)], ki_v)
        pltpu.async_copy(cnt.at[ki_v], cv_v, sem).wait()

        def div_body(j, _):
            cv = cv_v[pl.ds(j * 16, 16)]
            wv_v[pl.ds(j * 16, 16)] = 1.0 / cv
            return 0

        lax.fori_loop(0, _CH // 16, div_body, 0)
        pltpu.sync_copy(wv_v, w_hbm.at[pl.ds(e0, _CH)])
        return 0

    lax.fori_loop(0, _NCHW, w_body, 0)


@functools.partial(
    pl.kernel,
    out_type=jax.ShapeDtypeStruct((_NC, _ACC_PAD, _HH), jnp.float32),
    mesh=_sc_mesh,
    compiler_params=pltpu.CompilerParams(use_tc_tiling_on_sc=False),
    scratch_types=[
        pltpu.VMEM_SHARED((_ACC_PAD, _HH), jnp.float32),
        pltpu.VMEM((_CH,), jnp.int32),
        pltpu.VMEM((_CH,), jnp.int32),
        pltpu.VMEM((_CH, 16), jnp.float32),
        pltpu.VMEM((_CH, _HH), jnp.float32),
        pltpu.SemaphoreType.DMA,
    ],
)
def _edge_pass(hr_hbm, gi_hbm, si_hbm, w_hbm, za_hbm, out_hbm,
               acc, gi_v, si_v, w_v, rows_v, sem):
    c = lax.axis_index("c")
    s = lax.axis_index("s")
    pltpu.sync_copy(za_hbm, acc.at[pl.ds(s * _RPT, _RPT)])
    plsc.subcore_barrier()

    def chunk_body(ci, _):
        e0 = s * _EPT + ci * _CH
        pltpu.sync_copy(gi_hbm.at[pl.ds(e0, _CH)], gi_v)
        pltpu.sync_copy(si_hbm.at[pl.ds(e0, _CH)], si_v)
        pltpu.sync_copy(w_hbm.at[pl.ds(e0, _CH)], w_v)
        pltpu.async_copy(hr_hbm.at[c].at[gi_v], rows_v, sem).wait()

        def scale_body(i, _):
            wv = w_v[i, :]
            for k in range(_HH // 16):
                sl = pl.ds(k * 16, 16)
                rows_v[i, sl] = rows_v[i, sl] * wv
            return 0

        lax.fori_loop(0, _CH, scale_body, 0)
        pltpu.sync_copy(rows_v, acc.at[si_v], add=True)
        return 0

    lax.fori_loop(0, _NCH, chunk_body, 0)
    plsc.subcore_barrier()
    pltpu.sync_copy(acc.at[pl.ds(s * _RPT, _RPT)],
                    out_hbm.at[c].at[pl.ds(s * _RPT, _RPT)])


_BN = 400


def _proj_body(x_ref, w_ref, b_ref, o_ref):
    o_ref[...] = (jnp.dot(x_ref[...], w_ref[...],
                          preferred_element_type=jnp.float32) + b_ref[...])


def _proj(x, Win, b_in):
    return pl.pallas_call(
        _proj_body,
        grid=(_N // _BN,),
        in_specs=[
            pl.BlockSpec((_BN, _H), lambda i: (i, 0)),
            pl.BlockSpec((_H, _H), lambda i: (0, 0)),
            pl.BlockSpec((1, _H), lambda i: (0, 0)),
        ],
        out_specs=pl.BlockSpec((_BN, _H), lambda i: (i, 0)),
        out_shape=jax.ShapeDtypeStruct((_N, _H), jnp.float32),
    )(x, Win, b_in.reshape(1, _H))


def _hr_body(h_ref, w_ref, o_ref):
    o_ref[...] = jnp.dot(h_ref[...], w_ref[0, 0],
                         preferred_element_type=jnp.float32)[None, None]


_BNH = 2000


def _hr_halves(h, Wall_sp):
    return pl.pallas_call(
        _hr_body,
        grid=(_NC, _R, _N // _BNH),
        in_specs=[
            pl.BlockSpec((_BNH, _H), lambda c, r, i: (i, 0)),
            pl.BlockSpec((1, 1, _H, _HH), lambda c, r, i: (c, r, 0, 0)),
        ],
        out_specs=pl.BlockSpec((1, 1, _BNH, _HH), lambda c, r, i: (c, r, i, 0)),
        out_shape=jax.ShapeDtypeStruct((_NC, _R, _N, _HH), jnp.float32),
    )(h, Wall_sp)


def _fuse_body(first, s0_ref, s1_ref, t0_ref, t1_ref, h_ref, rs_ref, rt_ref,
               bs_ref, bt_ref, wft_ref, wfb_ref, bf_ref, g_ref, be_ref, o_ref):
    h = h_ref[...]
    hs = jnp.concatenate([s0_ref[...], s1_ref[...]], axis=-1) + \
        jnp.dot(h, rs_ref[...], preferred_element_type=jnp.float32) + bs_ref[...]
    ht = jnp.concatenate([t0_ref[...], t1_ref[...]], axis=-1) + \
        jnp.dot(h, rt_ref[...], preferred_element_type=jnp.float32) + bt_ref[...]
    hn = (jnp.dot(hs, wft_ref[...], preferred_element_type=jnp.float32)
          + jnp.dot(ht, wfb_ref[...], preferred_element_type=jnp.float32)
          + bf_ref[...])
    m = jnp.mean(hn, axis=-1, keepdims=True)
    d = hn - m
    v = jnp.mean(d * d, axis=-1, keepdims=True)
    hn = d * lax.rsqrt(v + 1e-5) * g_ref[...] + be_ref[...]
    hn = jnp.maximum(hn, 0.0)
    o_ref[...] = hn if first else h + hn


def _fuse(first, aggs0, aggs1, aggt0, aggt1, h, roots, roott, bias_s, bias_t,
          wf_top, wf_bot, bf_i, gamma_i, beta_i):
    row = lambda a: a.reshape(1, _H)
    half = pl.BlockSpec((_BN, _HH), lambda i: (i, 0))
    full = pl.BlockSpec((_BN, _H), lambda i: (i, 0))
    wspec = pl.BlockSpec((_H, _H), lambda i: (0, 0))
    bspec = pl.BlockSpec((1, _H), lambda i: (0, 0))
    return pl.pallas_call(
        functools.partial(_fuse_body, first),
        grid=(_N // _BN,),
        in_specs=[half, half, half, half, full, wspec, wspec, bspec, bspec,
                  wspec, wspec, bspec, bspec, bspec],
        out_specs=full,
        out_shape=jax.ShapeDtypeStruct((_N, _H), jnp.float32),
    )(aggs0, aggs1, aggt0, aggt1, h, roots, roott, row(bias_s), row(bias_t),
      wf_top, wf_bot, row(bf_i), row(gamma_i), row(beta_i))


def _head_body(h_ref, w1_ref, b1_ref, w2_ref, b2_ref, o_ref):
    t = jnp.maximum(jnp.dot(h_ref[...], w1_ref[...],
                            preferred_element_type=jnp.float32) + b1_ref[...], 0.0)
    o_ref[...] = jnp.dot(t, w2_ref[...],
                         preferred_element_type=jnp.float32) + b2_ref[...]


def _head(h, W1, b1, W2, b2):
    return pl.pallas_call(
        _head_body,
        grid=(_N // _BN,),
        in_specs=[
            pl.BlockSpec((_BN, _H), lambda i: (i, 0)),
            pl.BlockSpec((_H, _H), lambda i: (0, 0)),
            pl.BlockSpec((1, _H), lambda i: (0, 0)),
            pl.BlockSpec((_H, _O), lambda i: (0, 0)),
            pl.BlockSpec((1, _O), lambda i: (0, 0)),
        ],
        out_specs=pl.BlockSpec((_BN, _O), lambda i: (i, 0)),
        out_shape=jax.ShapeDtypeStruct((_N, _O), jnp.float32),
    )(h, W1, b1.reshape(1, _H), W2, b2.reshape(1, _O))


def kernel(x, edge_index, edge_type, Win, b_in, comp_s, basis_s, root_s,
           bias_s, comp_t, basis_t, root_t, bias_t, Wf, bf, gamma, beta,
           W1, b1, W2, b2):
    src = edge_index[0]
    dst = edge_index[1]
    et = edge_type

    keys = dst * _R + et
    gidx = et * _N + src
    sidx = dst + jnp.where(et < _RS, 0, _N)

    ones = jnp.ones((_CH,), jnp.float32)
    zc = jnp.zeros((_CNT_PT,), jnp.float32)
    za = jnp.zeros((_RPT, _HH), jnp.float32)

    w = _prep(keys, ones, zc)
    w16 = jnp.broadcast_to(w[:, None], (_E, 16))

    h = _proj(x, Win, b_in)
    for i in range(_L):
        Ws = jnp.einsum('rb,bio->rio', comp_s[i], basis_s[i])
        Wt = jnp.einsum('rb,bio->rio', comp_t[i], basis_t[i])
        Wall = jnp.concatenate([Ws, Wt], axis=0)
        Wall_sp = Wall.reshape(_R, _H, _NC, _HH).transpose(2, 0, 1, 3)
        hr = _hr_halves(h, Wall_sp).reshape(_NC, _R * _N, _HH)
        agg = _edge_pass(hr, gidx, sidx, w16, za)
        h = _fuse(i == 0,
                  agg[0, :_N], agg[1, :_N], agg[0, _N:], agg[1, _N:],
                  h, root_s[i], root_t[i], bias_s[i], bias_t[i],
                  Wf[i][:_H], Wf[i][_H:], bf[i], gamma[i], beta[i])
    return _head(h, W1, b1, W2, b2)


# speedup vs baseline: 21.6953x; 1.1557x over previous
import functools

import jax
import jax.numpy as jnp
from jax import lax
from jax.experimental import pallas as pl
from jax.experimental.pallas import tpu as pltpu
from jax.experimental.pallas import tpu_sc as plsc

_N = 10000
_E = 320000
_H = 128
_O = 32
_L = 3
_RS = 7
_RT = 3
_R = _RS + _RT

_NC = 2
_NS = 16
_CH = 200
_EPT = _E // _NS
_NCH = _EPT // _CH
_HH = _H // 2

_ACC_PAD = 20096
_RPT = _ACC_PAD // _NS
_CNT_PT = 6272
_CNT_ROWS = _CNT_PT * _NS
_EPW = _E // (_NC * _NS)
_NCHW = _EPW // _CH

_sc_mesh = plsc.VectorSubcoreMesh(core_axis_name="c", subcore_axis_name="s")


@functools.partial(
    pl.kernel,
    out_type=jax.ShapeDtypeStruct((_E,), jnp.float32),
    mesh=_sc_mesh,
    compiler_params=pltpu.CompilerParams(use_tc_tiling_on_sc=False),
    scratch_types=[
        pltpu.VMEM_SHARED((_CNT_ROWS,), jnp.float32),
        pltpu.VMEM((_CH,), jnp.int32),
        pltpu.VMEM((_CH,), jnp.float32),
        pltpu.VMEM((_CH,), jnp.float32),
        pltpu.VMEM((_CH,), jnp.float32),
        pltpu.SemaphoreType.DMA,
    ],
)
def _prep(keys_hbm, ones_hbm, zc_hbm, w_hbm,
          cnt, ki_v, ones_v, cv_v, wv_v, sem):
    c = lax.axis_index("c")
    s = lax.axis_index("s")
    pltpu.sync_copy(zc_hbm, cnt.at[pl.ds(s * _CNT_PT, _CNT_PT)])
    pltpu.sync_copy(ones_hbm, ones_v)
    plsc.subcore_barrier()

    def count_body(ci, _):
        e0 = s * _EPT + ci * _CH
        pltpu.sync_copy(keys_hbm.at[pl.ds(e0, _CH)], ki_v)
        pltpu.sync_copy(ones_v, cnt.at[ki_v], add=True)
        return 0

    lax.fori_loop(0, _NCH, count_body, 0)
    plsc.subcore_barrier()

    def w_body(ci, _):
        e0 = (c * _NS + s) * _EPW + ci * _CH
        pltpu.sync_copy(keys_hbm.at[pl.ds(e0, _CH)], ki_v)
        pltpu.async_copy(cnt.at[ki_v], cv_v, sem).wait()

        def div_body(j, _):
            cv = cv_v[pl.ds(j * 16, 16)]
            wv_v[pl.ds(j * 16, 16)] = 1.0 / cv
            return 0

        lax.fori_loop(0, _CH // 16, div_body, 0)
        pltpu.sync_copy(wv_v, w_hbm.at[pl.ds(e0, _CH)])
        return 0

    lax.fori_loop(0, _NCHW, w_body, 0)


@functools.partial(
    pl.kernel,
    out_type=jax.ShapeDtypeStruct((_NC, _ACC_PAD, _HH), jnp.float32),
    mesh=_sc_mesh,
    compiler_params=pltpu.CompilerParams(use_tc_tiling_on_sc=False),
    scratch_types=[
        pltpu.VMEM_SHARED((_ACC_PAD, _HH), jnp.float32),
        pltpu.VMEM((_CH,), jnp.int32),
        pltpu.VMEM((_CH,), jnp.int32),
        pltpu.VMEM((_CH,), jnp.int32),
        pltpu.VMEM((_CH,), jnp.int32),
        pltpu.VMEM((_CH, 16), jnp.float32),
        pltpu.VMEM((_CH, 16), jnp.float32),
        pltpu.VMEM((_CH, _HH), jnp.float32),
        pltpu.VMEM((_CH, _HH), jnp.float32),
        pltpu.SemaphoreType.DMA,
        pltpu.SemaphoreType.DMA,
        pltpu.SemaphoreType.DMA,
        pltpu.SemaphoreType.DMA,
        pltpu.SemaphoreType.DMA,
        pltpu.SemaphoreType.DMA,
        pltpu.SemaphoreType.DMA,
        pltpu.SemaphoreType.DMA,
    ],
)
def _edge_pass(hr_hbm, gi_hbm, si_hbm, w_hbm, za_hbm, out_hbm,
               acc, gi0_v, gi1_v, si0_v, si1_v, w0_v, w1_v, rows0_v, rows1_v,
               sem_gi0, sem_gi1, sem_si0, sem_si1, sem_w0, sem_w1,
               sem_g0, sem_g1):
    c = lax.axis_index("c")
    s = lax.axis_index("s")
    gi_b = (gi0_v, gi1_v)
    si_b = (si0_v, si1_v)
    w_b = (w0_v, w1_v)
    rows_b = (rows0_v, rows1_v)
    sem_gi = (sem_gi0, sem_gi1)
    sem_si = (sem_si0, sem_si1)
    sem_w = (sem_w0, sem_w1)
    sem_g = (sem_g0, sem_g1)

    pltpu.sync_copy(za_hbm, acc.at[pl.ds(s * _RPT, _RPT)])
    plsc.subcore_barrier()

    def start_idx(ci, b):
        e0 = s * _EPT + ci * _CH
        pltpu.async_copy(gi_hbm.at[pl.ds(e0, _CH)], gi_b[b], sem_gi[b])
        pltpu.async_copy(si_hbm.at[pl.ds(e0, _CH)], si_b[b], sem_si[b])
        pltpu.async_copy(w_hbm.at[pl.ds(e0, _CH)], w_b[b], sem_w[b])

    def wait_idx(ci, b):
        e0 = s * _EPT + ci * _CH
        pltpu.make_async_copy(gi_hbm.at[pl.ds(e0, _CH)], gi_b[b],
                              sem_gi[b]).wait()
        pltpu.make_async_copy(si_hbm.at[pl.ds(e0, _CH)], si_b[b],
                              sem_si[b]).wait()
        pltpu.make_async_copy(w_hbm.at[pl.ds(e0, _CH)], w_b[b],
                              sem_w[b]).wait()

    def start_gather(b):
        pltpu.async_copy(hr_hbm.at[c].at[gi_b[b]], rows_b[b], sem_g[b])

    def wait_gather(b):
        pltpu.make_async_copy(hr_hbm.at[c].at[gi_b[b]], rows_b[b],
                              sem_g[b]).wait()

    def scale(b):
        def scale_body(i, _):
            wv = w_b[b][i, :]
            for k in range(_HH // 16):
                sl = pl.ds(k * 16, 16)
                rows_b[b][i, sl] = rows_b[b][i, sl] * wv
            return 0

        lax.fori_loop(0, _CH, scale_body, 0)

    start_idx(0, 0)
    start_idx(1, 1)
    wait_idx(0, 0)
    start_gather(0)

    def pair_body(p, _):
        for b in (0, 1):
            ci = 2 * p + b
            nb = 1 - b
            wait_gather(b)
            wait_idx(ci + 1, nb)
            start_gather(nb)
            scale(b)
            pltpu.sync_copy(rows_b[b], acc.at[si_b[b]], add=True)
            start_idx(ci + 2, b)
        return 0

    lax.fori_loop(0, (_NCH - 2) // 2, pair_body, 0)
    wait_gather(0)
    wait_idx(_NCH - 1, 1)
    start_gather(1)
    scale(0)
    pltpu.sync_copy(rows_b[0], acc.at[si_b[0]], add=True)
    wait_gather(1)
    scale(1)
    pltpu.sync_copy(rows_b[1], acc.at[si_b[1]], add=True)
    plsc.subcore_barrier()
    pltpu.sync_copy(acc.at[pl.ds(s * _RPT, _RPT)],
                    out_hbm.at[c].at[pl.ds(s * _RPT, _RPT)])


_BN = 400


def _proj_body(x_ref, w_ref, b_ref, o_ref):
    o_ref[...] = (jnp.dot(x_ref[...], w_ref[...],
                          preferred_element_type=jnp.float32) + b_ref[...])


def _proj(x, Win, b_in):
    return pl.pallas_call(
        _proj_body,
        grid=(_N // _BN,),
        in_specs=[
            pl.BlockSpec((_BN, _H), lambda i: (i, 0)),
            pl.BlockSpec((_H, _H), lambda i: (0, 0)),
            pl.BlockSpec((1, _H), lambda i: (0, 0)),
        ],
        out_specs=pl.BlockSpec((_BN, _H), lambda i: (i, 0)),
        out_shape=jax.ShapeDtypeStruct((_N, _H), jnp.float32),
    )(x, Win, b_in.reshape(1, _H))


def _hr_body(h_ref, w_ref, o_ref):
    o_ref[...] = jnp.dot(h_ref[...], w_ref[0, 0],
                         preferred_element_type=jnp.float32)[None, None]


_BNH = 2000


def _hr_halves(h, Wall_sp):
    return pl.pallas_call(
        _hr_body,
        grid=(_NC, _R, _N // _BNH),
        in_specs=[
            pl.BlockSpec((_BNH, _H), lambda c, r, i: (i, 0)),
            pl.BlockSpec((1, 1, _H, _HH), lambda c, r, i: (c, r, 0, 0)),
        ],
        out_specs=pl.BlockSpec((1, 1, _BNH, _HH), lambda c, r, i: (c, r, i, 0)),
        out_shape=jax.ShapeDtypeStruct((_NC, _R, _N, _HH), jnp.float32),
    )(h, Wall_sp)


def _fuse_body(first, s0_ref, s1_ref, t0_ref, t1_ref, h_ref, rs_ref, rt_ref,
               bs_ref, bt_ref, wft_ref, wfb_ref, bf_ref, g_ref, be_ref, o_ref):
    h = h_ref[...]
    hs = jnp.concatenate([s0_ref[...], s1_ref[...]], axis=-1) + \
        jnp.dot(h, rs_ref[...], preferred_element_type=jnp.float32) + bs_ref[...]
    ht = jnp.concatenate([t0_ref[...], t1_ref[...]], axis=-1) + \
        jnp.dot(h, rt_ref[...], preferred_element_type=jnp.float32) + bt_ref[...]
    hn = (jnp.dot(hs, wft_ref[...], preferred_element_type=jnp.float32)
          + jnp.dot(ht, wfb_ref[...], preferred_element_type=jnp.float32)
          + bf_ref[...])
    m = jnp.mean(hn, axis=-1, keepdims=True)
    d = hn - m
    v = jnp.mean(d * d, axis=-1, keepdims=True)
    hn = d * lax.rsqrt(v + 1e-5) * g_ref[...] + be_ref[...]
    hn = jnp.maximum(hn, 0.0)
    o_ref[...] = hn if first else h + hn


def _fuse(first, aggs0, aggs1, aggt0, aggt1, h, roots, roott, bias_s, bias_t,
          wf_top, wf_bot, bf_i, gamma_i, beta_i):
    row = lambda a: a.reshape(1, _H)
    half = pl.BlockSpec((_BN, _HH), lambda i: (i, 0))
    full = pl.BlockSpec((_BN, _H), lambda i: (i, 0))
    wspec = pl.BlockSpec((_H, _H), lambda i: (0, 0))
    bspec = pl.BlockSpec((1, _H), lambda i: (0, 0))
    return pl.pallas_call(
        functools.partial(_fuse_body, first),
        grid=(_N // _BN,),
        in_specs=[half, half, half, half, full, wspec, wspec, bspec, bspec,
                  wspec, wspec, bspec, bspec, bspec],
        out_specs=full,
        out_shape=jax.ShapeDtypeStruct((_N, _H), jnp.float32),
    )(aggs0, aggs1, aggt0, aggt1, h, roots, roott, row(bias_s), row(bias_t),
      wf_top, wf_bot, row(bf_i), row(gamma_i), row(beta_i))


def _head_body(h_ref, w1_ref, b1_ref, w2_ref, b2_ref, o_ref):
    t = jnp.maximum(jnp.dot(h_ref[...], w1_ref[...],
                            preferred_element_type=jnp.float32) + b1_ref[...], 0.0)
    o_ref[...] = jnp.dot(t, w2_ref[...],
                         preferred_element_type=jnp.float32) + b2_ref[...]


def _head(h, W1, b1, W2, b2):
    return pl.pallas_call(
        _head_body,
        grid=(_N // _BN,),
        in_specs=[
            pl.BlockSpec((_BN, _H), lambda i: (i, 0)),
            pl.BlockSpec((_H, _H), lambda i: (0, 0)),
            pl.BlockSpec((1, _H), lambda i: (0, 0)),
            pl.BlockSpec((_H, _O), lambda i: (0, 0)),
            pl.BlockSpec((1, _O), lambda i: (0, 0)),
        ],
        out_specs=pl.BlockSpec((_BN, _O), lambda i: (i, 0)),
        out_shape=jax.ShapeDtypeStruct((_N, _O), jnp.float32),
    )(h, W1, b1.reshape(1, _H), W2, b2.reshape(1, _O))


def kernel(x, edge_index, edge_type, Win, b_in, comp_s, basis_s, root_s,
           bias_s, comp_t, basis_t, root_t, bias_t, Wf, bf, gamma, beta,
           W1, b1, W2, b2):
    src = edge_index[0]
    dst = edge_index[1]
    et = edge_type

    keys = dst * _R + et
    gidx = et * _N + src
    sidx = dst + jnp.where(et < _RS, 0, _N)

    ones = jnp.ones((_CH,), jnp.float32)
    zc = jnp.zeros((_CNT_PT,), jnp.float32)
    za = jnp.zeros((_RPT, _HH), jnp.float32)

    w = _prep(keys, ones, zc)
    w16 = jnp.broadcast_to(w[:, None], (_E, 16))

    h = _proj(x, Win, b_in)
    for i in range(_L):
        Ws = jnp.einsum('rb,bio->rio', comp_s[i], basis_s[i])
        Wt = jnp.einsum('rb,bio->rio', comp_t[i], basis_t[i])
        Wall = jnp.concatenate([Ws, Wt], axis=0)
        Wall_sp = Wall.reshape(_R, _H, _NC, _HH).transpose(2, 0, 1, 3)
        hr = _hr_halves(h, Wall_sp).reshape(_NC, _R * _N, _HH)
        agg = _edge_pass(hr, gidx, sidx, w16, za)
        h = _fuse(i == 0,
                  agg[0, :_N], agg[1, :_N], agg[0, _N:], agg[1, _N:],
                  h, root_s[i], root_t[i], bias_s[i], bias_t[i],
                  Wf[i][:_H], Wf[i][_H:], bf[i], gamma[i], beta[i])
    return _head(h, W1, b1, W2, b2)


# speedup vs baseline: 24.4710x; 1.1279x over previous
import functools

import jax
import jax.numpy as jnp
from jax import lax
from jax.experimental import pallas as pl
from jax.experimental.pallas import tpu as pltpu
from jax.experimental.pallas import tpu_sc as plsc

_N = 10000
_E = 320000
_H = 128
_O = 32
_L = 3
_RS = 7
_RT = 3
_R = _RS + _RT

_NC = 2
_NS = 16
_CH = 200
_EPT = _E // _NS
_NCH = _EPT // _CH
_HH = _H // 2

_ACC_PAD = 20096
_RPT = _ACC_PAD // _NS
_CNT_PT = 6272
_CNT_ROWS = _CNT_PT * _NS
_EPW = _E // (_NC * _NS)
_PCH = 2000
_NPCH = _EPT // _PCH
_NPCHW = _EPW // _PCH

_sc_mesh = plsc.VectorSubcoreMesh(core_axis_name="c", subcore_axis_name="s")


@functools.partial(
    pl.kernel,
    out_type=jax.ShapeDtypeStruct((_E,), jnp.float32),
    mesh=_sc_mesh,
    compiler_params=pltpu.CompilerParams(use_tc_tiling_on_sc=False),
    scratch_types=[
        pltpu.VMEM_SHARED((_CNT_ROWS,), jnp.float32),
        pltpu.VMEM((_PCH,), jnp.int32),
        pltpu.VMEM((_PCH,), jnp.float32),
        pltpu.VMEM((_PCH,), jnp.float32),
        pltpu.VMEM((_PCH,), jnp.float32),
        pltpu.SemaphoreType.DMA,
    ],
)
def _prep(keys_hbm, ones_hbm, zc_hbm, w_hbm,
          cnt, ki_v, ones_v, cv_v, wv_v, sem):
    c = lax.axis_index("c")
    s = lax.axis_index("s")
    pltpu.sync_copy(zc_hbm, cnt.at[pl.ds(s * _CNT_PT, _CNT_PT)])
    pltpu.sync_copy(ones_hbm, ones_v)
    plsc.subcore_barrier()

    def count_body(ci, _):
        e0 = s * _EPT + ci * _PCH
        pltpu.sync_copy(keys_hbm.at[pl.ds(e0, _PCH)], ki_v)
        pltpu.sync_copy(ones_v, cnt.at[ki_v], add=True)
        return 0

    lax.fori_loop(0, _NPCH, count_body, 0)
    plsc.subcore_barrier()

    def w_body(ci, _):
        e0 = (c * _NS + s) * _EPW + ci * _PCH
        pltpu.sync_copy(keys_hbm.at[pl.ds(e0, _PCH)], ki_v)
        pltpu.async_copy(cnt.at[ki_v], cv_v, sem).wait()

        def div_body(j, _):
            cv = cv_v[pl.ds(j * 16, 16)]
            wv_v[pl.ds(j * 16, 16)] = 1.0 / cv
            return 0

        lax.fori_loop(0, _PCH // 16, div_body, 0)
        pltpu.sync_copy(wv_v, w_hbm.at[pl.ds(e0, _PCH)])
        return 0

    lax.fori_loop(0, _NPCHW, w_body, 0)


@functools.partial(
    pl.kernel,
    out_type=jax.ShapeDtypeStruct((_NC, _ACC_PAD, _HH), jnp.float32),
    mesh=_sc_mesh,
    compiler_params=pltpu.CompilerParams(use_tc_tiling_on_sc=False),
    scratch_types=[
        pltpu.VMEM_SHARED((_ACC_PAD, _HH), jnp.float32),
        pltpu.VMEM((_CH,), jnp.int32),
        pltpu.VMEM((_CH,), jnp.int32),
        pltpu.VMEM((_CH,), jnp.int32),
        pltpu.VMEM((_CH,), jnp.int32),
        pltpu.VMEM((_CH * 16,), jnp.float32),
        pltpu.VMEM((_CH * 16,), jnp.float32),
        pltpu.VMEM((_CH, _HH), jnp.float32),
        pltpu.VMEM((_CH, _HH), jnp.float32),
        pltpu.SemaphoreType.DMA,
        pltpu.SemaphoreType.DMA,
        pltpu.SemaphoreType.DMA,
        pltpu.SemaphoreType.DMA,
        pltpu.SemaphoreType.DMA,
        pltpu.SemaphoreType.DMA,
        pltpu.SemaphoreType.DMA,
        pltpu.SemaphoreType.DMA,
    ],
)
def _edge_pass(hr_hbm, gi_hbm, si_hbm, w_hbm, za_hbm, out_hbm,
               acc, gi0_v, gi1_v, si0_v, si1_v, w0_v, w1_v, rows0_v, rows1_v,
               sem_gi0, sem_gi1, sem_si0, sem_si1, sem_w0, sem_w1,
               sem_g0, sem_g1):
    c = lax.axis_index("c")
    s = lax.axis_index("s")
    gi_b = (gi0_v, gi1_v)
    si_b = (si0_v, si1_v)
    w_b = (w0_v, w1_v)
    rows_b = (rows0_v, rows1_v)
    sem_gi = (sem_gi0, sem_gi1)
    sem_si = (sem_si0, sem_si1)
    sem_w = (sem_w0, sem_w1)
    sem_g = (sem_g0, sem_g1)

    pltpu.sync_copy(za_hbm, acc.at[pl.ds(s * _RPT, _RPT)])
    plsc.subcore_barrier()

    def start_idx(ci, b):
        e0 = s * _EPT + ci * _CH
        pltpu.async_copy(gi_hbm.at[pl.ds(e0, _CH)], gi_b[b], sem_gi[b])
        pltpu.async_copy(si_hbm.at[pl.ds(e0, _CH)], si_b[b], sem_si[b])
        pltpu.async_copy(w_hbm.at[pl.ds(e0 * 16, _CH * 16)], w_b[b],
                         sem_w[b])

    def wait_idx(ci, b):
        e0 = s * _EPT + ci * _CH
        pltpu.make_async_copy(gi_hbm.at[pl.ds(e0, _CH)], gi_b[b],
                              sem_gi[b]).wait()
        pltpu.make_async_copy(si_hbm.at[pl.ds(e0, _CH)], si_b[b],
                              sem_si[b]).wait()
        pltpu.make_async_copy(w_hbm.at[pl.ds(e0 * 16, _CH * 16)], w_b[b],
                              sem_w[b]).wait()

    def start_gather(b):
        pltpu.async_copy(hr_hbm.at[c].at[gi_b[b]], rows_b[b], sem_g[b])

    def wait_gather(b):
        pltpu.make_async_copy(hr_hbm.at[c].at[gi_b[b]], rows_b[b],
                              sem_g[b]).wait()

    def scale(b):
        def scale_body(i, _):
            wv = w_b[b][pl.ds(i * 16, 16)]
            for k in range(_HH // 16):
                sl = pl.ds(k * 16, 16)
                rows_b[b][i, sl] = rows_b[b][i, sl] * wv
            return 0

        lax.fori_loop(0, _CH, scale_body, 0)

    start_idx(0, 0)
    start_idx(1, 1)
    wait_idx(0, 0)
    start_gather(0)

    def pair_body(p, _):
        for b in (0, 1):
            ci = 2 * p + b
            nb = 1 - b
            wait_gather(b)
            wait_idx(ci + 1, nb)
            start_gather(nb)
            scale(b)
            pltpu.sync_copy(rows_b[b], acc.at[si_b[b]], add=True)
            start_idx(ci + 2, b)
        return 0

    lax.fori_loop(0, (_NCH - 2) // 2, pair_body, 0)
    wait_gather(0)
    wait_idx(_NCH - 1, 1)
    start_gather(1)
    scale(0)
    pltpu.sync_copy(rows_b[0], acc.at[si_b[0]], add=True)
    wait_gather(1)
    scale(1)
    pltpu.sync_copy(rows_b[1], acc.at[si_b[1]], add=True)
    plsc.subcore_barrier()
    pltpu.sync_copy(acc.at[pl.ds(s * _RPT, _RPT)],
                    out_hbm.at[c].at[pl.ds(s * _RPT, _RPT)])


_BN = 400


def _proj_body(x_ref, w_ref, b_ref, o_ref):
    o_ref[...] = (jnp.dot(x_ref[...], w_ref[...],
                          preferred_element_type=jnp.float32) + b_ref[...])


def _proj(x, Win, b_in):
    return pl.pallas_call(
        _proj_body,
        grid=(_N // _BN,),
        in_specs=[
            pl.BlockSpec((_BN, _H), lambda i: (i, 0)),
            pl.BlockSpec((_H, _H), lambda i: (0, 0)),
            pl.BlockSpec((1, _H), lambda i: (0, 0)),
        ],
        out_specs=pl.BlockSpec((_BN, _H), lambda i: (i, 0)),
        out_shape=jax.ShapeDtypeStruct((_N, _H), jnp.float32),
    )(x, Win, b_in.reshape(1, _H))


def _hr_body(h_ref, w_ref, o_ref):
    h = h_ref[...]
    for r in range(_R):
        o_ref[0, r] = jnp.dot(h, w_ref[0, r],
                              preferred_element_type=jnp.float32)


_BNH = 2000


def _hr_halves(h, Wall_sp):
    return pl.pallas_call(
        _hr_body,
        grid=(_NC, _N // _BNH),
        in_specs=[
            pl.BlockSpec((_BNH, _H), lambda c, i: (i, 0)),
            pl.BlockSpec((1, _R, _H, _HH), lambda c, i: (c, 0, 0, 0)),
        ],
        out_specs=pl.BlockSpec((1, _R, _BNH, _HH), lambda c, i: (c, 0, i, 0)),
        out_shape=jax.ShapeDtypeStruct((_NC, _R, _N, _HH), jnp.float32),
    )(h, Wall_sp)


def _fuse_body(first, s0_ref, s1_ref, t0_ref, t1_ref, h_ref, rs_ref, rt_ref,
               bs_ref, bt_ref, wft_ref, wfb_ref, bf_ref, g_ref, be_ref, o_ref):
    h = h_ref[...]
    hs = jnp.concatenate([s0_ref[...], s1_ref[...]], axis=-1) + \
        jnp.dot(h, rs_ref[...], preferred_element_type=jnp.float32) + bs_ref[...]
    ht = jnp.concatenate([t0_ref[...], t1_ref[...]], axis=-1) + \
        jnp.dot(h, rt_ref[...], preferred_element_type=jnp.float32) + bt_ref[...]
    hn = (jnp.dot(hs, wft_ref[...], preferred_element_type=jnp.float32)
          + jnp.dot(ht, wfb_ref[...], preferred_element_type=jnp.float32)
          + bf_ref[...])
    m = jnp.mean(hn, axis=-1, keepdims=True)
    d = hn - m
    v = jnp.mean(d * d, axis=-1, keepdims=True)
    hn = d * lax.rsqrt(v + 1e-5) * g_ref[...] + be_ref[...]
    hn = jnp.maximum(hn, 0.0)
    o_ref[...] = hn if first else h + hn


def _fuse(first, aggs0, aggs1, aggt0, aggt1, h, roots, roott, bias_s, bias_t,
          wf_top, wf_bot, bf_i, gamma_i, beta_i):
    row = lambda a: a.reshape(1, _H)
    half = pl.BlockSpec((_BN, _HH), lambda i: (i, 0))
    full = pl.BlockSpec((_BN, _H), lambda i: (i, 0))
    wspec = pl.BlockSpec((_H, _H), lambda i: (0, 0))
    bspec = pl.BlockSpec((1, _H), lambda i: (0, 0))
    return pl.pallas_call(
        functools.partial(_fuse_body, first),
        grid=(_N // _BN,),
        in_specs=[half, half, half, half, full, wspec, wspec, bspec, bspec,
                  wspec, wspec, bspec, bspec, bspec],
        out_specs=full,
        out_shape=jax.ShapeDtypeStruct((_N, _H), jnp.float32),
    )(aggs0, aggs1, aggt0, aggt1, h, roots, roott, row(bias_s), row(bias_t),
      wf_top, wf_bot, row(bf_i), row(gamma_i), row(beta_i))


def _head_body(h_ref, w1_ref, b1_ref, w2_ref, b2_ref, o_ref):
    t = jnp.maximum(jnp.dot(h_ref[...], w1_ref[...],
                            preferred_element_type=jnp.float32) + b1_ref[...], 0.0)
    o_ref[...] = jnp.dot(t, w2_ref[...],
                         preferred_element_type=jnp.float32) + b2_ref[...]


def _head(h, W1, b1, W2, b2):
    return pl.pallas_call(
        _head_body,
        grid=(_N // _BN,),
        in_specs=[
            pl.BlockSpec((_BN, _H), lambda i: (i, 0)),
            pl.BlockSpec((_H, _H), lambda i: (0, 0)),
            pl.BlockSpec((1, _H), lambda i: (0, 0)),
            pl.BlockSpec((_H, _O), lambda i: (0, 0)),
            pl.BlockSpec((1, _O), lambda i: (0, 0)),
        ],
        out_specs=pl.BlockSpec((_BN, _O), lambda i: (i, 0)),
        out_shape=jax.ShapeDtypeStruct((_N, _O), jnp.float32),
    )(h, W1, b1.reshape(1, _H), W2, b2.reshape(1, _O))


def kernel(x, edge_index, edge_type, Win, b_in, comp_s, basis_s, root_s,
           bias_s, comp_t, basis_t, root_t, bias_t, Wf, bf, gamma, beta,
           W1, b1, W2, b2):
    src = edge_index[0]
    dst = edge_index[1]
    et = edge_type

    keys = dst * _R + et
    gidx = et * _N + src
    sidx = dst + jnp.where(et < _RS, 0, _N)

    ones = jnp.ones((_PCH,), jnp.float32)
    zc = jnp.zeros((_CNT_PT,), jnp.float32)
    za = jnp.zeros((_RPT, _HH), jnp.float32)

    w = _prep(keys, ones, zc)
    w16 = jnp.broadcast_to(w[:, None], (_E, 16)).reshape(_E * 16)

    h = _proj(x, Win, b_in)
    for i in range(_L):
        Ws = jnp.einsum('rb,bio->rio', comp_s[i], basis_s[i])
        Wt = jnp.einsum('rb,bio->rio', comp_t[i], basis_t[i])
        Wall = jnp.concatenate([Ws, Wt], axis=0)
        Wall_sp = Wall.reshape(_R, _H, _NC, _HH).transpose(2, 0, 1, 3)
        hr = _hr_halves(h, Wall_sp).reshape(_NC, _R * _N, _HH)
        agg = _edge_pass(hr, gidx, sidx, w16, za)
        h = _fuse(i == 0,
                  agg[0, :_N], agg[1, :_N], agg[0, _N:], agg[1, _N:],
                  h, root_s[i], root_t[i], bias_s[i], bias_t[i],
                  Wf[i][:_H], Wf[i][_H:], bf[i], gamma[i], beta[i])
    return _head(h, W1, b1, W2, b2)


# speedup vs baseline: 26.8284x; 1.0963x over previous
import functools

import jax
import jax.numpy as jnp
from jax import lax
from jax.experimental import pallas as pl
from jax.experimental.pallas import tpu as pltpu
from jax.experimental.pallas import tpu_sc as plsc

_N = 10000
_E = 320000
_H = 128
_O = 32
_L = 3
_RS = 7
_RT = 3
_R = _RS + _RT

_NC = 2
_NS = 16
_CH = 200
_EPT = _E // _NS
_NCH = _EPT // _CH
_HH = _H // 2

_ACC_PAD = 20096
_RPT = _ACC_PAD // _NS
_CNT_PT = 6272
_CNT_ROWS = _CNT_PT * _NS
_EPW = _E // (_NC * _NS)
_PCH = 2000
_NPCH = _EPT // _PCH
_NPCHW = _EPW // _PCH

_sc_mesh = plsc.VectorSubcoreMesh(core_axis_name="c", subcore_axis_name="s")


@functools.partial(
    pl.kernel,
    out_type=jax.ShapeDtypeStruct((_E,), jnp.float32),
    mesh=_sc_mesh,
    compiler_params=pltpu.CompilerParams(use_tc_tiling_on_sc=False),
    scratch_types=[
        pltpu.VMEM_SHARED((_CNT_ROWS,), jnp.float32),
        pltpu.VMEM((_PCH,), jnp.int32),
        pltpu.VMEM((_PCH,), jnp.float32),
        pltpu.VMEM((_PCH,), jnp.float32),
        pltpu.VMEM((_PCH,), jnp.float32),
        pltpu.SemaphoreType.DMA,
    ],
)
def _prep(keys_hbm, ones_hbm, zc_hbm, w_hbm,
          cnt, ki_v, ones_v, cv_v, wv_v, sem):
    c = lax.axis_index("c")
    s = lax.axis_index("s")
    pltpu.sync_copy(zc_hbm, cnt.at[pl.ds(s * _CNT_PT, _CNT_PT)])
    pltpu.sync_copy(ones_hbm, ones_v)
    plsc.subcore_barrier()

    def count_body(ci, _):
        e0 = s * _EPT + ci * _PCH
        pltpu.sync_copy(keys_hbm.at[pl.ds(e0, _PCH)], ki_v)
        pltpu.sync_copy(ones_v, cnt.at[ki_v], add=True)
        return 0

    lax.fori_loop(0, _NPCH, count_body, 0)
    plsc.subcore_barrier()

    def w_body(ci, _):
        e0 = (c * _NS + s) * _EPW + ci * _PCH
        pltpu.sync_copy(keys_hbm.at[pl.ds(e0, _PCH)], ki_v)
        pltpu.async_copy(cnt.at[ki_v], cv_v, sem).wait()

        def div_body(j, _):
            cv = cv_v[pl.ds(j * 16, 16)]
            wv_v[pl.ds(j * 16, 16)] = 1.0 / cv
            return 0

        lax.fori_loop(0, _PCH // 16, div_body, 0)
        pltpu.sync_copy(wv_v, w_hbm.at[pl.ds(e0, _PCH)])
        return 0

    lax.fori_loop(0, _NPCHW, w_body, 0)


@functools.partial(
    pl.kernel,
    out_type=jax.ShapeDtypeStruct((_NC, _ACC_PAD, _HH), jnp.float32),
    mesh=_sc_mesh,
    compiler_params=pltpu.CompilerParams(use_tc_tiling_on_sc=False),
    scratch_types=[
        pltpu.VMEM_SHARED((_ACC_PAD, _HH), jnp.float32),
        pltpu.VMEM((_CH,), jnp.int32),
        pltpu.VMEM((_CH,), jnp.int32),
        pltpu.VMEM((_CH,), jnp.int32),
        pltpu.VMEM((_CH,), jnp.int32),
        pltpu.VMEM((_CH,), jnp.int32),
        pltpu.VMEM((_CH,), jnp.int32),
        pltpu.VMEM((_CH * 16,), jnp.float32),
        pltpu.VMEM((_CH * 16,), jnp.float32),
        pltpu.VMEM((_CH, _HH), jnp.float32),
        pltpu.VMEM((_CH, _HH), jnp.float32),
        pltpu.SemaphoreType.DMA,
        pltpu.SemaphoreType.DMA,
        pltpu.SemaphoreType.DMA,
        pltpu.SemaphoreType.DMA,
        pltpu.SemaphoreType.DMA,
        pltpu.SemaphoreType.DMA,
        pltpu.SemaphoreType.DMA,
        pltpu.SemaphoreType.DMA,
        pltpu.SemaphoreType.DMA,
        pltpu.SemaphoreType.DMA,
        pltpu.SemaphoreType.DMA,
        pltpu.SemaphoreType.DMA,
        pltpu.SemaphoreType.DMA,
        pltpu.SemaphoreType.DMA,
    ],
)
def _edge_pass(hr_hbm, gi_hbm, si_hbm, w_hbm, za_hbm, out_hbm,
               acc, gi0_v, gi1_v, si0_v, si1_v, si2_v, si3_v, w0_v, w1_v,
               rows0_v, rows1_v,
               sem_gi0, sem_gi1, sem_si0, sem_si1, sem_si2, sem_si3,
               sem_w0, sem_w1, sem_g0, sem_g1,
               sem_s0, sem_s1, sem_s2, sem_s3):
    c = lax.axis_index("c")
    s = lax.axis_index("s")
    gi_b = (gi0_v, gi1_v)
    si_q = (si0_v, si1_v, si2_v, si3_v)
    w_b = (w0_v, w1_v)
    rows_b = (rows0_v, rows1_v)
    sem_gi = (sem_gi0, sem_gi1)
    sem_si = (sem_si0, sem_si1, sem_si2, sem_si3)
    sem_w = (sem_w0, sem_w1)
    sem_g = (sem_g0, sem_g1)
    sem_s = (sem_s0, sem_s1, sem_s2, sem_s3)

    pltpu.sync_copy(za_hbm, acc.at[pl.ds(s * _RPT, _RPT)])
    plsc.subcore_barrier()

    def start_idx(ci, b, q):
        e0 = s * _EPT + ci * _CH
        pltpu.async_copy(gi_hbm.at[pl.ds(e0, _CH)], gi_b[b], sem_gi[b])
        pltpu.async_copy(si_hbm.at[pl.ds(e0, _CH)], si_q[q], sem_si[q])
        pltpu.async_copy(w_hbm.at[pl.ds(e0 * 16, _CH * 16)], w_b[b],
                         sem_w[b])

    def wait_idx(ci, b, q):
        e0 = s * _EPT + ci * _CH
        pltpu.make_async_copy(gi_hbm.at[pl.ds(e0, _CH)], gi_b[b],
                              sem_gi[b]).wait()
        pltpu.make_async_copy(si_hbm.at[pl.ds(e0, _CH)], si_q[q],
                              sem_si[q]).wait()
        pltpu.make_async_copy(w_hbm.at[pl.ds(e0 * 16, _CH * 16)], w_b[b],
                              sem_w[b]).wait()

    def start_gather(b):
        pltpu.async_copy(hr_hbm.at[c].at[gi_b[b]], rows_b[b], sem_g[b])

    def wait_gather(b):
        pltpu.make_async_copy(hr_hbm.at[c].at[gi_b[b]], rows_b[b],
                              sem_g[b]).wait()

    def start_scatter(b, q):
        pltpu.async_copy(rows_b[b], acc.at[si_q[q]], sem_s[q], add=True)

    def wait_scatter(b, q):
        pltpu.make_async_copy(rows_b[b], acc.at[si_q[q]],
                              sem_s[q]).wait()

    def scale(b):
        def scale_body(i, _):
            wv = w_b[b][pl.ds(i * 16, 16)]
            for k in range(_HH // 16):
                sl = pl.ds(k * 16, 16)
                rows_b[b][i, sl] = rows_b[b][i, sl] * wv
            return 0

        lax.fori_loop(0, _CH, scale_body, 0)

    def body(ci, b, q, first=False, g_next=True, i_next=True):
        wait_gather(b)
        if not first:
            wait_scatter(1 - b, (q + 3) % 4)
        if g_next:
            wait_idx(ci + 1, 1 - b, (q + 1) % 4)
            start_gather(1 - b)
        scale(b)
        start_scatter(b, q)
        if i_next:
            start_idx(ci + 2, b, (q + 2) % 4)

    start_idx(0, 0, 0)
    start_idx(1, 1, 1)
    wait_idx(0, 0, 0)
    start_gather(0)

    body(0, 0, 0, first=True)
    body(1, 1, 1)
    body(2, 0, 2)
    body(3, 1, 3)

    def quad_body(p, _):
        for j in range(4):
            body(4 * p + j, j % 2, j)
        return 0

    lax.fori_loop(1, (_NCH - 4) // 4, quad_body, 0)

    body(_NCH - 4, 0, 0)
    body(_NCH - 3, 1, 1)
    body(_NCH - 2, 0, 2, i_next=False)
    body(_NCH - 1, 1, 3, g_next=False, i_next=False)
    wait_scatter(1, 3)

    plsc.subcore_barrier()
    pltpu.sync_copy(acc.at[pl.ds(s * _RPT, _RPT)],
                    out_hbm.at[c].at[pl.ds(s * _RPT, _RPT)])


_BN = 400


def _proj_body(x_ref, w_ref, b_ref, o_ref):
    o_ref[...] = (jnp.dot(x_ref[...], w_ref[...],
                          preferred_element_type=jnp.float32) + b_ref[...])


def _proj(x, Win, b_in):
    return pl.pallas_call(
        _proj_body,
        grid=(_N // _BN,),
        in_specs=[
            pl.BlockSpec((_BN, _H), lambda i: (i, 0)),
            pl.BlockSpec((_H, _H), lambda i: (0, 0)),
            pl.BlockSpec((1, _H), lambda i: (0, 0)),
        ],
        out_specs=pl.BlockSpec((_BN, _H), lambda i: (i, 0)),
        out_shape=jax.ShapeDtypeStruct((_N, _H), jnp.float32),
    )(x, Win, b_in.reshape(1, _H))


def _hr_body(h_ref, w_ref, o_ref):
    h = h_ref[...]
    for r in range(_R):
        o_ref[0, r] = jnp.dot(h, w_ref[0, r],
                              preferred_element_type=jnp.float32)


_BNH = 2000


def _hr_halves(h, Wall_sp):
    return pl.pallas_call(
        _hr_body,
        grid=(_NC, _N // _BNH),
        in_specs=[
            pl.BlockSpec((_BNH, _H), lambda c, i: (i, 0)),
            pl.BlockSpec((1, _R, _H, _HH), lambda c, i: (c, 0, 0, 0)),
        ],
        out_specs=pl.BlockSpec((1, _R, _BNH, _HH), lambda c, i: (c, 0, i, 0)),
        out_shape=jax.ShapeDtypeStruct((_NC, _R, _N, _HH), jnp.float32),
    )(h, Wall_sp)


def _fuse_body(first, s0_ref, s1_ref, t0_ref, t1_ref, h_ref, rs_ref, rt_ref,
               bs_ref, bt_ref, wft_ref, wfb_ref, bf_ref, g_ref, be_ref, o_ref):
    h = h_ref[...]
    hs = jnp.concatenate([s0_ref[...], s1_ref[...]], axis=-1) + \
        jnp.dot(h, rs_ref[...], preferred_element_type=jnp.float32) + bs_ref[...]
    ht = jnp.concatenate([t0_ref[...], t1_ref[...]], axis=-1) + \
        jnp.dot(h, rt_ref[...], preferred_element_type=jnp.float32) + bt_ref[...]
    hn = (jnp.dot(hs, wft_ref[...], preferred_element_type=jnp.float32)
          + jnp.dot(ht, wfb_ref[...], preferred_element_type=jnp.float32)
          + bf_ref[...])
    m = jnp.mean(hn, axis=-1, keepdims=True)
    d = hn - m
    v = jnp.mean(d * d, axis=-1, keepdims=True)
    hn = d * lax.rsqrt(v + 1e-5) * g_ref[...] + be_ref[...]
    hn = jnp.maximum(hn, 0.0)
    o_ref[...] = hn if first else h + hn


def _fuse(first, aggs0, aggs1, aggt0, aggt1, h, roots, roott, bias_s, bias_t,
          wf_top, wf_bot, bf_i, gamma_i, beta_i):
    row = lambda a: a.reshape(1, _H)
    half = pl.BlockSpec((_BN, _HH), lambda i: (i, 0))
    full = pl.BlockSpec((_BN, _H), lambda i: (i, 0))
    wspec = pl.BlockSpec((_H, _H), lambda i: (0, 0))
    bspec = pl.BlockSpec((1, _H), lambda i: (0, 0))
    return pl.pallas_call(
        functools.partial(_fuse_body, first),
        grid=(_N // _BN,),
        in_specs=[half, half, half, half, full, wspec, wspec, bspec, bspec,
                  wspec, wspec, bspec, bspec, bspec],
        out_specs=full,
        out_shape=jax.ShapeDtypeStruct((_N, _H), jnp.float32),
    )(aggs0, aggs1, aggt0, aggt1, h, roots, roott, row(bias_s), row(bias_t),
      wf_top, wf_bot, row(bf_i), row(gamma_i), row(beta_i))


def _head_body(h_ref, w1_ref, b1_ref, w2_ref, b2_ref, o_ref):
    t = jnp.maximum(jnp.dot(h_ref[...], w1_ref[...],
                            preferred_element_type=jnp.float32) + b1_ref[...], 0.0)
    o_ref[...] = jnp.dot(t, w2_ref[...],
                         preferred_element_type=jnp.float32) + b2_ref[...]


def _head(h, W1, b1, W2, b2):
    return pl.pallas_call(
        _head_body,
        grid=(_N // _BN,),
        in_specs=[
            pl.BlockSpec((_BN, _H), lambda i: (i, 0)),
            pl.BlockSpec((_H, _H), lambda i: (0, 0)),
            pl.BlockSpec((1, _H), lambda i: (0, 0)),
            pl.BlockSpec((_H, _O), lambda i: (0, 0)),
            pl.BlockSpec((1, _O), lambda i: (0, 0)),
        ],
        out_specs=pl.BlockSpec((_BN, _O), lambda i: (i, 0)),
        out_shape=jax.ShapeDtypeStruct((_N, _O), jnp.float32),
    )(h, W1, b1.reshape(1, _H), W2, b2.reshape(1, _O))


def kernel(x, edge_index, edge_type, Win, b_in, comp_s, basis_s, root_s,
           bias_s, comp_t, basis_t, root_t, bias_t, Wf, bf, gamma, beta,
           W1, b1, W2, b2):
    src = edge_index[0]
    dst = edge_index[1]
    et = edge_type

    keys = dst * _R + et
    gidx = et * _N + src
    sidx = dst + jnp.where(et < _RS, 0, _N)

    ones = jnp.ones((_PCH,), jnp.float32)
    zc = jnp.zeros((_CNT_PT,), jnp.float32)
    za = jnp.zeros((_RPT, _HH), jnp.float32)

    w = _prep(keys, ones, zc)
    w16 = jnp.broadcast_to(w[:, None], (_E, 16)).reshape(_E * 16)

    h = _proj(x, Win, b_in)
    for i in range(_L):
        Ws = jnp.einsum('rb,bio->rio', comp_s[i], basis_s[i])
        Wt = jnp.einsum('rb,bio->rio', comp_t[i], basis_t[i])
        Wall = jnp.concatenate([Ws, Wt], axis=0)
        Wall_sp = Wall.reshape(_R, _H, _NC, _HH).transpose(2, 0, 1, 3)
        hr = _hr_halves(h, Wall_sp).reshape(_NC, _R * _N, _HH)
        agg = _edge_pass(hr, gidx, sidx, w16, za)
        h = _fuse(i == 0,
                  agg[0, :_N], agg[1, :_N], agg[0, _N:], agg[1, _N:],
                  h, root_s[i], root_t[i], bias_s[i], bias_t[i],
                  Wf[i][:_H], Wf[i][_H:], bf[i], gamma[i], beta[i])
    return _head(h, W1, b1, W2, b2)


# speedup vs baseline: 27.8191x; 1.0369x over previous
import functools

import jax
import jax.numpy as jnp
from jax import lax
from jax.experimental import pallas as pl
from jax.experimental.pallas import tpu as pltpu
from jax.experimental.pallas import tpu_sc as plsc

_N = 10000
_E = 320000
_H = 128
_O = 32
_L = 3
_RS = 7
_RT = 3
_R = _RS + _RT

_NC = 2
_NS = 16
_CH = 200
_EPT = _E // _NS
_NCH = _EPT // _CH
_HH = _H // 2

_ACC_PAD = 20096
_RPT = _ACC_PAD // _NS
_CNT_PT = 6272
_CNT_ROWS = _CNT_PT * _NS
_EPW = _E // (_NC * _NS)
_PCH = 2000
_NPCH = _EPT // _PCH
_NPCHW = _EPW // _PCH

_sc_mesh = plsc.VectorSubcoreMesh(core_axis_name="c", subcore_axis_name="s")


@functools.partial(
    pl.kernel,
    out_type=jax.ShapeDtypeStruct((_E,), jnp.float32),
    mesh=_sc_mesh,
    compiler_params=pltpu.CompilerParams(use_tc_tiling_on_sc=False),
    scratch_types=[
        pltpu.VMEM_SHARED((_CNT_ROWS,), jnp.float32),
        pltpu.VMEM((_PCH,), jnp.int32),
        pltpu.VMEM((_PCH,), jnp.float32),
        pltpu.VMEM((_PCH,), jnp.float32),
        pltpu.VMEM((_PCH,), jnp.float32),
        pltpu.SemaphoreType.DMA,
    ],
)
def _prep(keys_hbm, ones_hbm, zc_hbm, w_hbm,
          cnt, ki_v, ones_v, cv_v, wv_v, sem):
    c = lax.axis_index("c")
    s = lax.axis_index("s")
    pltpu.sync_copy(zc_hbm, cnt.at[pl.ds(s * _CNT_PT, _CNT_PT)])
    pltpu.sync_copy(ones_hbm, ones_v)
    plsc.subcore_barrier()

    def count_body(ci, _):
        e0 = s * _EPT + ci * _PCH
        pltpu.sync_copy(keys_hbm.at[pl.ds(e0, _PCH)], ki_v)
        pltpu.sync_copy(ones_v, cnt.at[ki_v], add=True)
        return 0

    lax.fori_loop(0, _NPCH, count_body, 0)
    plsc.subcore_barrier()

    def w_body(ci, _):
        e0 = (c * _NS + s) * _EPW + ci * _PCH
        pltpu.sync_copy(keys_hbm.at[pl.ds(e0, _PCH)], ki_v)
        pltpu.async_copy(cnt.at[ki_v], cv_v, sem).wait()

        def div_body(j, _):
            cv = cv_v[pl.ds(j * 16, 16)]
            wv_v[pl.ds(j * 16, 16)] = 1.0 / cv
            return 0

        lax.fori_loop(0, _PCH // 16, div_body, 0)
        pltpu.sync_copy(wv_v, w_hbm.at[pl.ds(e0, _PCH)])
        return 0

    lax.fori_loop(0, _NPCHW, w_body, 0)


@functools.partial(
    pl.kernel,
    out_type=jax.ShapeDtypeStruct((_NC, _ACC_PAD, _HH), jnp.float32),
    mesh=_sc_mesh,
    compiler_params=pltpu.CompilerParams(use_tc_tiling_on_sc=False),
    scratch_types=[
        pltpu.VMEM_SHARED((_ACC_PAD, _HH), jnp.float32),
        pltpu.VMEM((_CH,), jnp.int32),
        pltpu.VMEM((_CH,), jnp.int32),
        pltpu.VMEM((_CH,), jnp.int32),
        pltpu.VMEM((_CH,), jnp.int32),
        pltpu.VMEM((_CH,), jnp.int32),
        pltpu.VMEM((_CH,), jnp.int32),
        pltpu.VMEM((_CH * 16,), jnp.float32),
        pltpu.VMEM((_CH * 16,), jnp.float32),
        pltpu.VMEM((_CH, _HH), jnp.float32),
        pltpu.VMEM((_CH, _HH), jnp.float32),
        pltpu.SemaphoreType.DMA,
        pltpu.SemaphoreType.DMA,
        pltpu.SemaphoreType.DMA,
        pltpu.SemaphoreType.DMA,
        pltpu.SemaphoreType.DMA,
        pltpu.SemaphoreType.DMA,
        pltpu.SemaphoreType.DMA,
        pltpu.SemaphoreType.DMA,
        pltpu.SemaphoreType.DMA,
        pltpu.SemaphoreType.DMA,
        pltpu.SemaphoreType.DMA,
        pltpu.SemaphoreType.DMA,
        pltpu.SemaphoreType.DMA,
        pltpu.SemaphoreType.DMA,
    ],
)
def _edge_pass(hr_hbm, gi_hbm, si_hbm, w_hbm, za_hbm, out_hbm,
               acc, gi0_v, gi1_v, si0_v, si1_v, si2_v, si3_v, w0_v, w1_v,
               rows0_v, rows1_v,
               sem_gi0, sem_gi1, sem_si0, sem_si1, sem_si2, sem_si3,
               sem_w0, sem_w1, sem_g0, sem_g1,
               sem_s0, sem_s1, sem_s2, sem_s3):
    c = lax.axis_index("c")
    s = lax.axis_index("s")
    gi_b = (gi0_v, gi1_v)
    si_q = (si0_v, si1_v, si2_v, si3_v)
    w_b = (w0_v, w1_v)
    rows_b = (rows0_v, rows1_v)
    sem_gi = (sem_gi0, sem_gi1)
    sem_si = (sem_si0, sem_si1, sem_si2, sem_si3)
    sem_w = (sem_w0, sem_w1)
    sem_g = (sem_g0, sem_g1)
    sem_s = (sem_s0, sem_s1, sem_s2, sem_s3)

    pltpu.sync_copy(za_hbm, acc.at[pl.ds(s * _RPT, _RPT)])
    plsc.subcore_barrier()

    def start_idx(ci, b, q):
        e0 = s * _EPT + ci * _CH
        pltpu.async_copy(gi_hbm.at[pl.ds(e0, _CH)], gi_b[b], sem_gi[b])
        pltpu.async_copy(si_hbm.at[pl.ds(e0, _CH)], si_q[q], sem_si[q])
        pltpu.async_copy(w_hbm.at[pl.ds(e0 * 16, _CH * 16)], w_b[b],
                         sem_w[b])

    def wait_idx(ci, b, q):
        e0 = s * _EPT + ci * _CH
        pltpu.make_async_copy(gi_hbm.at[pl.ds(e0, _CH)], gi_b[b],
                              sem_gi[b]).wait()
        pltpu.make_async_copy(si_hbm.at[pl.ds(e0, _CH)], si_q[q],
                              sem_si[q]).wait()
        pltpu.make_async_copy(w_hbm.at[pl.ds(e0 * 16, _CH * 16)], w_b[b],
                              sem_w[b]).wait()

    def start_gather(b):
        pltpu.async_copy(hr_hbm.at[c].at[gi_b[b]], rows_b[b], sem_g[b])

    def wait_gather(b):
        pltpu.make_async_copy(hr_hbm.at[c].at[gi_b[b]], rows_b[b],
                              sem_g[b]).wait()

    def start_scatter(b, q):
        pltpu.async_copy(rows_b[b], acc.at[si_q[q]], sem_s[q], add=True)

    def wait_scatter(b, q):
        pltpu.make_async_copy(rows_b[b], acc.at[si_q[q]],
                              sem_s[q]).wait()

    def scale(b):
        def scale_body(i4, _):
            for u in range(4):
                i = i4 * 4 + u
                wv = w_b[b][pl.ds(i * 16, 16)]
                for k in range(_HH // 16):
                    sl = pl.ds(k * 16, 16)
                    rows_b[b][i, sl] = rows_b[b][i, sl] * wv
            return 0

        lax.fori_loop(0, _CH // 4, scale_body, 0)

    def body(ci, b, q, first=False, g_next=True, i_next=True):
        wait_gather(b)
        if not first:
            wait_scatter(1 - b, (q + 3) % 4)
        if g_next:
            wait_idx(ci + 1, 1 - b, (q + 1) % 4)
            start_gather(1 - b)
        scale(b)
        start_scatter(b, q)
        if i_next:
            start_idx(ci + 2, b, (q + 2) % 4)

    start_idx(0, 0, 0)
    start_idx(1, 1, 1)
    wait_idx(0, 0, 0)
    start_gather(0)

    body(0, 0, 0, first=True)
    body(1, 1, 1)
    body(2, 0, 2)
    body(3, 1, 3)

    def quad_body(p, _):
        for j in range(4):
            body(4 * p + j, j % 2, j)
        return 0

    lax.fori_loop(1, (_NCH - 4) // 4, quad_body, 0)

    body(_NCH - 4, 0, 0)
    body(_NCH - 3, 1, 1)
    body(_NCH - 2, 0, 2, i_next=False)
    body(_NCH - 1, 1, 3, g_next=False, i_next=False)
    wait_scatter(1, 3)

    plsc.subcore_barrier()
    pltpu.sync_copy(acc.at[pl.ds(s * _RPT, _RPT)],
                    out_hbm.at[c].at[pl.ds(s * _RPT, _RPT)])


_BN = 400


def _proj_body(x_ref, w_ref, b_ref, o_ref):
    o_ref[...] = (jnp.dot(x_ref[...], w_ref[...],
                          preferred_element_type=jnp.float32) + b_ref[...])


def _proj(x, Win, b_in):
    return pl.pallas_call(
        _proj_body,
        grid=(_N // _BN,),
        in_specs=[
            pl.BlockSpec((_BN, _H), lambda i: (i, 0)),
            pl.BlockSpec((_H, _H), lambda i: (0, 0)),
            pl.BlockSpec((1, _H), lambda i: (0, 0)),
        ],
        out_specs=pl.BlockSpec((_BN, _H), lambda i: (i, 0)),
        out_shape=jax.ShapeDtypeStruct((_N, _H), jnp.float32),
    )(x, Win, b_in.reshape(1, _H))


def _hr_body(h_ref, w_ref, o_ref):
    h = h_ref[...]
    for r in range(_R):
        o_ref[0, r] = jnp.dot(h, w_ref[0, r],
                              preferred_element_type=jnp.float32)


_BNH = 2000


def _hr_halves(h, Wall_sp):
    return pl.pallas_call(
        _hr_body,
        grid=(_NC, _N // _BNH),
        in_specs=[
            pl.BlockSpec((_BNH, _H), lambda c, i: (i, 0)),
            pl.BlockSpec((1, _R, _H, _HH), lambda c, i: (c, 0, 0, 0)),
        ],
        out_specs=pl.BlockSpec((1, _R, _BNH, _HH), lambda c, i: (c, 0, i, 0)),
        out_shape=jax.ShapeDtypeStruct((_NC, _R, _N, _HH), jnp.float32),
    )(h, Wall_sp)


def _fuse_body(first, s0_ref, s1_ref, t0_ref, t1_ref, h_ref, rs_ref, rt_ref,
               bs_ref, bt_ref, wft_ref, wfb_ref, bf_ref, g_ref, be_ref, o_ref):
    h = h_ref[...]
    hs = jnp.concatenate([s0_ref[...], s1_ref[...]], axis=-1) + \
        jnp.dot(h, rs_ref[...], preferred_element_type=jnp.float32) + bs_ref[...]
    ht = jnp.concatenate([t0_ref[...], t1_ref[...]], axis=-1) + \
        jnp.dot(h, rt_ref[...], preferred_element_type=jnp.float32) + bt_ref[...]
    hn = (jnp.dot(hs, wft_ref[...], preferred_element_type=jnp.float32)
          + jnp.dot(ht, wfb_ref[...], preferred_element_type=jnp.float32)
          + bf_ref[...])
    m = jnp.mean(hn, axis=-1, keepdims=True)
    d = hn - m
    v = jnp.mean(d * d, axis=-1, keepdims=True)
    hn = d * lax.rsqrt(v + 1e-5) * g_ref[...] + be_ref[...]
    hn = jnp.maximum(hn, 0.0)
    o_ref[...] = hn if first else h + hn


def _fuse(first, aggs0, aggs1, aggt0, aggt1, h, roots, roott, bias_s, bias_t,
          wf_top, wf_bot, bf_i, gamma_i, beta_i):
    row = lambda a: a.reshape(1, _H)
    half = pl.BlockSpec((_BN, _HH), lambda i: (i, 0))
    full = pl.BlockSpec((_BN, _H), lambda i: (i, 0))
    wspec = pl.BlockSpec((_H, _H), lambda i: (0, 0))
    bspec = pl.BlockSpec((1, _H), lambda i: (0, 0))
    return pl.pallas_call(
        functools.partial(_fuse_body, first),
        grid=(_N // _BN,),
        in_specs=[half, half, half, half, full, wspec, wspec, bspec, bspec,
                  wspec, wspec, bspec, bspec, bspec],
        out_specs=full,
        out_shape=jax.ShapeDtypeStruct((_N, _H), jnp.float32),
    )(aggs0, aggs1, aggt0, aggt1, h, roots, roott, row(bias_s), row(bias_t),
      wf_top, wf_bot, row(bf_i), row(gamma_i), row(beta_i))


def _head_body(h_ref, w1_ref, b1_ref, w2_ref, b2_ref, o_ref):
    t = jnp.maximum(jnp.dot(h_ref[...], w1_ref[...],
                            preferred_element_type=jnp.float32) + b1_ref[...], 0.0)
    o_ref[...] = jnp.dot(t, w2_ref[...],
                         preferred_element_type=jnp.float32) + b2_ref[...]


def _head(h, W1, b1, W2, b2):
    return pl.pallas_call(
        _head_body,
        grid=(_N // _BN,),
        in_specs=[
            pl.BlockSpec((_BN, _H), lambda i: (i, 0)),
            pl.BlockSpec((_H, _H), lambda i: (0, 0)),
            pl.BlockSpec((1, _H), lambda i: (0, 0)),
            pl.BlockSpec((_H, _O), lambda i: (0, 0)),
            pl.BlockSpec((1, _O), lambda i: (0, 0)),
        ],
        out_specs=pl.BlockSpec((_BN, _O), lambda i: (i, 0)),
        out_shape=jax.ShapeDtypeStruct((_N, _O), jnp.float32),
    )(h, W1, b1.reshape(1, _H), W2, b2.reshape(1, _O))


def kernel(x, edge_index, edge_type, Win, b_in, comp_s, basis_s, root_s,
           bias_s, comp_t, basis_t, root_t, bias_t, Wf, bf, gamma, beta,
           W1, b1, W2, b2):
    src = edge_index[0]
    dst = edge_index[1]
    et = edge_type

    keys = dst * _R + et
    gidx = et * _N + src
    sidx = dst + jnp.where(et < _RS, 0, _N)

    ones = jnp.ones((_PCH,), jnp.float32)
    zc = jnp.zeros((_CNT_PT,), jnp.float32)
    za = jnp.zeros((_RPT, _HH), jnp.float32)

    w = _prep(keys, ones, zc)
    w16 = jnp.broadcast_to(w[:, None], (_E, 16)).reshape(_E * 16)

    h = _proj(x, Win, b_in)
    for i in range(_L):
        Ws = jnp.einsum('rb,bio->rio', comp_s[i], basis_s[i])
        Wt = jnp.einsum('rb,bio->rio', comp_t[i], basis_t[i])
        Wall = jnp.concatenate([Ws, Wt], axis=0)
        Wall_sp = Wall.reshape(_R, _H, _NC, _HH).transpose(2, 0, 1, 3)
        hr = _hr_halves(h, Wall_sp).reshape(_NC, _R * _N, _HH)
        agg = _edge_pass(hr, gidx, sidx, w16, za)
        h = _fuse(i == 0,
                  agg[0, :_N], agg[1, :_N], agg[0, _N:], agg[1, _N:],
                  h, root_s[i], root_t[i], bias_s[i], bias_t[i],
                  Wf[i][:_H], Wf[i][_H:], bf[i], gamma[i], beta[i])
    return _head(h, W1, b1, W2, b2)
